# baseline (speedup 1.0000x reference)
import functools

import jax
import jax.numpy as jnp
from jax import lax
from jax.experimental import pallas as pl
from jax.experimental.pallas import tpu as pltpu
from jax.experimental.pallas import tpu_sc as plsc

B = 4096
L = 2048
D = 32
V = 5
VP = 8

NC = 2
NS = 16
NW = NC * NS
RPW = B // NW

QR = 4
NQ = RPW // QR

LANES = 16
SEG = 128
NSEG = L // SEG
DT = D // 8

_mesh = plsc.VectorSubcoreMesh(core_axis_name="c", subcore_axis_name="s")


@functools.partial(
    pl.kernel,
    out_type=jax.ShapeDtypeStruct((B, DT, NSEG, 8, SEG), jnp.float32),
    mesh=_mesh,
    scratch_types=[
        pltpu.VMEM((QR, L), jnp.int32),
        pltpu.VMEM((D, VP), jnp.float32),
        pltpu.VMEM((DT, 8, L), jnp.float32),
        pltpu.VMEM((QR, DT, 8, SEG), jnp.float32),
        pltpu.VMEM((QR, DT, 8, SEG), jnp.float32),
        pltpu.SemaphoreType.DMA,
        pltpu.SemaphoreType.DMA,
    ],
    compiler_params=pltpu.CompilerParams(
        use_tc_tiling_on_sc=False, needs_layout_passes=False),
)
def _sc_lookup(semT_hbm, posT_hbm, x_hbm, out_hbm, xv, semT, posT, tb0, tb1,
               osem0, osem1):
    c = lax.axis_index("c")
    s = lax.axis_index("s")
    wid = s * NC + c
    base = wid * RPW

    pltpu.sync_copy(semT_hbm, semT)
    pltpu.sync_copy(posT_hbm, posT)

    def build_seg(seg, tb):
        def group(g, carry):
            off = seg * SEG + g * LANES
            xs = [xv[rr, pl.ds(off, LANES)] for rr in range(QR)]
            for d in range(D):
                pv = posT[d // 8, d % 8, pl.ds(off, LANES)]
                for rr in range(QR):
                    sv = plsc.load_gather(
                        semT, [jnp.full((LANES,), d, jnp.int32), xs[rr]])
                    tb[rr, d // 8, d % 8, pl.ds(g * LANES, LANES)] = sv + pv
            return carry

        lax.fori_loop(0, SEG // LANES, group, 0, unroll=False)

    def quad_body(q, carry):
        row0 = base + q * QR
        pltpu.sync_copy(x_hbm.at[pl.ds(row0, QR)], xv)

        def seg_pair(p, carry2):
            for par, tb, osem in ((0, tb0, osem0), (1, tb1, osem1)):
                seg = 2 * p + par

                @pl.when(jnp.logical_or(q > 0, p > 0))
                def _drain_out():
                    pltpu.make_async_copy(
                        tb, out_hbm.at[pl.ds(row0, QR), :, seg], osem).wait()

                build_seg(seg, tb)
                pltpu.async_copy(
                    tb, out_hbm.at[pl.ds(row0, QR), :, seg], osem)
            return carry2

        lax.fori_loop(0, NSEG // 2, seg_pair, 0, unroll=False)
        return carry

    lax.fori_loop(0, NQ, quad_body, 0, unroll=False)
    pltpu.make_async_copy(tb0, out_hbm.at[pl.ds(base, QR), :, 0], osem0).wait()
    pltpu.make_async_copy(tb1, out_hbm.at[pl.ds(base, QR), :, 1], osem1).wait()


def kernel(x, semantic_table, positional_table):
    semT = jnp.pad(semantic_table.T, ((0, 0), (0, VP - V)))
    posT = positional_table.T.reshape(DT, 8, L)
    out5 = _sc_lookup(semT, posT, x.astype(jnp.int32))
    return out5.transpose(0, 2, 4, 1, 3).reshape(B, L, D)
